# Initial kernel scaffold; baseline (speedup 1.0000x reference)
import jax
import jax.numpy as jnp
from jax.experimental import pallas as pl


def kernel(x, edge_index, edge_type, params):
    raise NotImplementedError("write your pallas kernel here")


# speedup vs baseline: 4.4478x; 4.4478x over previous
import functools

import jax
import jax.numpy as jnp
import numpy as np
from jax import lax
from jax.experimental import pallas as pl
from jax.experimental.pallas import tpu as pltpu
from jax.experimental.pallas import tpu_sc as plsc

N = 10000
E = 320000
D = 128
H = 4
DH = D // H
NT = 16
NC = 1
NS = 16
B = 32
G = B // 16
NBLK = E // NS // B
BN = 1000
CPT = 10
ROWS_T = N // CPT


def _tc_proj(h, Wa, Wb):
    def body(x_ref, wa_ref, wb_ref, qq_ref, kv_ref):
        x = x_ref[...]
        qq_ref[...] = jnp.dot(x, wa_ref[...], preferred_element_type=jnp.float32)
        kv_ref[...] = jnp.dot(x, wb_ref[...], preferred_element_type=jnp.float32)

    return pl.pallas_call(
        body,
        grid=(N // BN,),
        in_specs=[
            pl.BlockSpec((BN, D), lambda i: (i, 0)),
            pl.BlockSpec((D, D + NT * H), lambda i: (0, 0)),
            pl.BlockSpec((D, 2 * D), lambda i: (0, 0)),
        ],
        out_specs=[
            pl.BlockSpec((BN, D + NT * H), lambda i: (i, 0)),
            pl.BlockSpec((BN, 2 * D), lambda i: (i, 0)),
        ],
        out_shape=[
            jax.ShapeDtypeStruct((N, D + NT * H), jnp.float32),
            jax.ShapeDtypeStruct((N, 2 * D), jnp.float32),
        ],
    )(h, Wa, Wb)


def _tc_merge(h, agg2, den2, E4, WnT, WnB, bn, ln_s, ln_b):
    def body(x_ref, a_ref, den_ref, e4_ref, wnt_ref, wnb_ref,
             bn_ref, lns_ref, lnb_ref, o_ref):
        A = a_ref[0]
        denx = jnp.dot(den_ref[0], e4_ref[...],
                       preferred_element_type=jnp.float32)
        aggn = A / (denx + 1e-9)
        z = (jnp.dot(x_ref[...], wnt_ref[...], preferred_element_type=jnp.float32)
             + jnp.dot(aggn, wnb_ref[...], preferred_element_type=jnp.float32)
             + bn_ref[...])
        z = jnp.maximum(z, 0.0)
        mu = jnp.mean(z, axis=1, keepdims=True)
        var = jnp.mean((z - mu) ** 2, axis=1, keepdims=True)
        o_ref[...] = (z - mu) / jnp.sqrt(var + 1e-5) * lns_ref[...] + lnb_ref[...]

    return pl.pallas_call(
        body,
        grid=(N // BN,),
        in_specs=[
            pl.BlockSpec((BN, D), lambda i: (i, 0)),
            pl.BlockSpec((NC, BN, D), lambda i: (0, i, 0)),
            pl.BlockSpec((NC, BN, 16), lambda i: (0, i, 0)),
            pl.BlockSpec((16, D), lambda i: (0, 0)),
            pl.BlockSpec((D, D), lambda i: (0, 0)),
            pl.BlockSpec((D, D), lambda i: (0, 0)),
            pl.BlockSpec((1, D), lambda i: (0, 0)),
            pl.BlockSpec((1, D), lambda i: (0, 0)),
            pl.BlockSpec((1, D), lambda i: (0, 0)),
        ],
        out_specs=pl.BlockSpec((BN, D), lambda i: (i, 0)),
        out_shape=jax.ShapeDtypeStruct((N, D), jnp.float32),
    )(h, agg2, den2, E4, WnT, WnB, bn, ln_s, ln_b)


def _sc_body(qq_hbm, kv_hbm, ev_hbm, src_hbm, dst_hbm, ty_hbm,
             agg_out, den_out,
             src_v, dst_v, ty_v, qq_v, kv_v, srow_v, wrow_v, ev_v,
             agg_s, den_s, sem0, sem1):
    s = lax.axis_index("s")
    zvec = jnp.zeros((16,), jnp.float32)
    lane = lax.iota(jnp.int32, 16)

    pltpu.sync_copy(ev_hbm, ev_v)

    @pl.when(s < CPT)
    def _():
        def zrow(i, c):
            for j in range(D // 16):
                srow_v[i, pl.ds(j * 16, 16)] = zvec
            return c
        lax.fori_loop(0, B, zrow, 0)
        def zw(i, c):
            wrow_v[i, pl.ds(0, 16)] = zvec
            return c
        lax.fori_loop(0, B, zw, 0)
        base = s * ROWS_T
        def zcpy(i, c):
            pltpu.sync_copy(srow_v, agg_s.at[pl.ds(base + i * B, B), :])
            pltpu.sync_copy(wrow_v, den_s.at[pl.ds(base + i * B, B), :])
            return c
        lax.fori_loop(0, ROWS_T // B, zcpy, 0)
        rem = ROWS_T % B
        pltpu.sync_copy(srow_v.at[pl.ds(0, rem), :],
                        agg_s.at[pl.ds(base + (ROWS_T // B) * B, rem), :])
        pltpu.sync_copy(wrow_v.at[pl.ds(0, rem), :],
                        den_s.at[pl.ds(base + (ROWS_T // B) * B, rem), :])

    plsc.subcore_barrier()

    def do_block(base):
        pltpu.sync_copy(src_hbm.at[pl.ds(base, B)], src_v)
        pltpu.sync_copy(dst_hbm.at[pl.ds(base, B)], dst_v)
        pltpu.sync_copy(ty_hbm.at[pl.ds(base, B)], ty_v)
        cp1 = pltpu.async_copy(qq_hbm.at[dst_v], qq_v, sem0)
        cp2 = pltpu.async_copy(kv_hbm.at[src_v], kv_v, sem1)
        cp1.wait()
        cp2.wait()

        def group(eg, carry2):
            evec = eg * 16 + lane
            tvec = ty_v[pl.ds(eg * 16, 16)]
            one = jnp.ones((16,), jnp.int32)
            accs = [plsc.load_gather(qq_v, [evec, D + tvec * H + hh])
                    for hh in range(H)]
            dv = jnp.zeros((16,), jnp.int32)
            for dcol in range(D):
                qv = plsc.load_gather(qq_v, [evec, dv])
                kvv = plsc.load_gather(kv_v, [evec, dv])
                accs[dcol // DH] = accs[dcol // DH] + qv * kvv
                dv = dv + one
            ws = [jnp.exp(a) for a in accs]
            for hh in range(H):
                plsc.store_scatter(wrow_v, [evec, jnp.full((16,), hh, jnp.int32)],
                                   ws[hh])
            sv = jnp.zeros((16,), jnp.int32)
            for dcol in range(D):
                vv = (plsc.load_gather(kv_v, [evec, sv + D])
                      + plsc.load_gather(ev_v, [tvec, sv]))
                plsc.store_scatter(srow_v, [evec, sv], ws[dcol // DH] * vv)
                sv = sv + one
            return carry2

        lax.fori_loop(0, G, group, 0)
        pltpu.sync_copy(srow_v, agg_s.at[dst_v], add=True)
        pltpu.sync_copy(wrow_v, den_s.at[dst_v], add=True)

    def block(g, carry):
        do_block(s * NBLK * B + g * B)
        return carry

    lax.fori_loop(0, NBLK, block, 0)
    plsc.subcore_barrier()

    @pl.when(s < CPT)
    def _():
        pltpu.sync_copy(agg_s.at[pl.ds(s * ROWS_T, ROWS_T), :],
                        agg_out.at[0, pl.ds(s * ROWS_T, ROWS_T), :])
        pltpu.sync_copy(den_s.at[pl.ds(s * ROWS_T, ROWS_T), :],
                        den_out.at[0, pl.ds(s * ROWS_T, ROWS_T), :])


_sc_edge = functools.partial(
    pl.kernel,
    out_type=(
        jax.ShapeDtypeStruct((NC, N, D), jnp.float32),
        jax.ShapeDtypeStruct((NC, N, 16), jnp.float32),
    ),
    mesh=plsc.VectorSubcoreMesh(core_axis_name="c", subcore_axis_name="s",
                                num_cores=NC),
    compiler_params=pltpu.CompilerParams(use_tc_tiling_on_sc=False,
                                         needs_layout_passes=False),
    scratch_types=[
        pltpu.VMEM((B,), jnp.int32),
        pltpu.VMEM((B,), jnp.int32),
        pltpu.VMEM((B,), jnp.int32),
        pltpu.VMEM((B, D + NT * H), jnp.float32),
        pltpu.VMEM((B, 2 * D), jnp.float32),
        pltpu.VMEM((B, D), jnp.float32),
        pltpu.VMEM((B, 16), jnp.float32),
        pltpu.VMEM((NT, D), jnp.float32),
        pltpu.VMEM_SHARED((N, D), jnp.float32),
        pltpu.VMEM_SHARED((N, 16), jnp.float32),
        pltpu.SemaphoreType.DMA,
        pltpu.SemaphoreType.DMA,
    ],
)(_sc_body)


def kernel(x, edge_index, edge_type, params):
    src = edge_index[0]
    dst = edge_index[1]
    ty = edge_type
    heads = jnp.arange(D) // DH
    hid = jnp.arange(H)
    E4 = (jnp.arange(16)[:, None] == heads[None, :]).astype(jnp.float32)

    h = x
    for p in params['layers']:
        scale = 1.0 / np.sqrt(DH)
        EkT = params['edge_table'] @ p['Wek']
        EvT = params['edge_table'] @ p['Wev']
        M = (EkT.T[:, :, None]
             * (heads[:, None, None] == hid[None, None, :])).reshape(D, NT * H)
        Wq_s = p['Wq'] * scale
        Wa = jnp.concatenate([Wq_s, Wq_s @ M], axis=1)
        Wb = jnp.concatenate([p['Wk'], p['Wv']], axis=1)

        qq, kv = _tc_proj(h, Wa, Wb)
        agg2, den2 = _sc_edge(qq, kv, EvT, src, dst, ty)
        h = _tc_merge(h, agg2, den2, E4,
                      p['Wn'][:D], p['Wn'][D:],
                      p['bn'].reshape(1, D),
                      p['ln_s'].reshape(1, D),
                      p['ln_b'].reshape(1, D))
    return h.reshape(N, 1, D)


# speedup vs baseline: 8.6234x; 1.9388x over previous
import functools

import jax
import jax.numpy as jnp
import numpy as np
from jax import lax
from jax.experimental import pallas as pl
from jax.experimental.pallas import tpu as pltpu
from jax.experimental.pallas import tpu_sc as plsc

N = 10000
E = 320000
D = 128
H = 4
DH = D // H
NT = 16
NC = 2
NS = 16
NW = NC * NS
EPW = E // NW
B = 16
G = B // 16
NBLK = EPW // B
SW = D + 16
BN = 1000
CPT = 10
ROWS_T = N // CPT


def _tc_proj(h, Wa, Wb):
    def body(x_ref, wa_ref, wb_ref, qq_ref, kv_ref):
        x = x_ref[...]
        qq_ref[...] = jnp.dot(x, wa_ref[...], preferred_element_type=jnp.float32)
        kv_ref[...] = jnp.dot(x, wb_ref[...], preferred_element_type=jnp.float32)

    return pl.pallas_call(
        body,
        grid=(N // BN,),
        in_specs=[
            pl.BlockSpec((BN, D), lambda i: (i, 0)),
            pl.BlockSpec((D, D + NT * H), lambda i: (0, 0)),
            pl.BlockSpec((D, 2 * D), lambda i: (0, 0)),
        ],
        out_specs=[
            pl.BlockSpec((BN, D + NT * H), lambda i: (i, 0)),
            pl.BlockSpec((BN, 2 * D), lambda i: (i, 0)),
        ],
        out_shape=[
            jax.ShapeDtypeStruct((N, D + NT * H), jnp.float32),
            jax.ShapeDtypeStruct((N, 2 * D), jnp.float32),
        ],
    )(h, Wa, Wb)


def _tc_merge(h, agg2, E4, WnT, WnB, bn, ln_s, ln_b):
    def body(x_ref, a_ref, e4_ref, wnt_ref, wnb_ref,
             bn_ref, lns_ref, lnb_ref, o_ref):
        ab = a_ref[0] + a_ref[1]
        A = ab[:, :D]
        denx = jnp.dot(ab[:, D:], e4_ref[...],
                       preferred_element_type=jnp.float32)
        aggn = A / (denx + 1e-9)
        z = (jnp.dot(x_ref[...], wnt_ref[...], preferred_element_type=jnp.float32)
             + jnp.dot(aggn, wnb_ref[...], preferred_element_type=jnp.float32)
             + bn_ref[...])
        z = jnp.maximum(z, 0.0)
        mu = jnp.mean(z, axis=1, keepdims=True)
        var = jnp.mean((z - mu) ** 2, axis=1, keepdims=True)
        o_ref[...] = (z - mu) / jnp.sqrt(var + 1e-5) * lns_ref[...] + lnb_ref[...]

    return pl.pallas_call(
        body,
        grid=(N // BN,),
        in_specs=[
            pl.BlockSpec((BN, D), lambda i: (i, 0)),
            pl.BlockSpec((NC, BN, SW), lambda i: (0, i, 0)),
            pl.BlockSpec((16, D), lambda i: (0, 0)),
            pl.BlockSpec((D, D), lambda i: (0, 0)),
            pl.BlockSpec((D, D), lambda i: (0, 0)),
            pl.BlockSpec((1, D), lambda i: (0, 0)),
            pl.BlockSpec((1, D), lambda i: (0, 0)),
            pl.BlockSpec((1, D), lambda i: (0, 0)),
        ],
        out_specs=pl.BlockSpec((BN, D), lambda i: (i, 0)),
        out_shape=jax.ShapeDtypeStruct((N, D), jnp.float32),
    )(h, agg2, E4, WnT, WnB, bn, ln_s, ln_b)


def _sc_body(qq_hbm, kv_hbm, ev_hbm, src_hbm, dst_hbm, ty_hbm,
             agg_out,
             src_v0, src_v1, dst_v0, dst_v1, ty_v0, ty_v1,
             sdst_v0, sdst_v1,
             qq_v0, qq_v1, kv_v0, kv_v1,
             srow_v0, srow_v1, ev_v,
             agg_s,
             isem0, isem1, gsem0, gsem1, ssem0, ssem1):
    c = lax.axis_index("c")
    s = lax.axis_index("s")
    wid = c * NS + s
    zvec = jnp.zeros((16,), jnp.float32)
    lane = lax.iota(jnp.int32, 16)
    one = jnp.ones((16,), jnp.int32)

    srcs = (src_v0, src_v1)
    dsts = (dst_v0, dst_v1)
    tys = (ty_v0, ty_v1)
    sdsts = (sdst_v0, sdst_v1)
    qqs = (qq_v0, qq_v1)
    kvs = (kv_v0, kv_v1)
    srows = (srow_v0, srow_v1)
    isems = (isem0, isem1)
    gsems = (gsem0, gsem1)
    ssems = (ssem0, ssem1)

    pltpu.sync_copy(ev_hbm, ev_v)

    @pl.when(s < CPT)
    def _():
        def zrow(i, cy):
            for j in range(SW // 16):
                srow_v0[i, pl.ds(j * 16, 16)] = zvec
            return cy
        lax.fori_loop(0, B, zrow, 0)
        base = s * ROWS_T
        def zcpy(i, cy):
            pltpu.sync_copy(srow_v0, agg_s.at[pl.ds(base + i * B, B), :])
            return cy
        lax.fori_loop(0, ROWS_T // B, zcpy, 0)
        rem = ROWS_T % B
        pltpu.sync_copy(srow_v0.at[pl.ds(0, rem), :],
                        agg_s.at[pl.ds(base + (ROWS_T // B) * B, rem), :])

    plsc.subcore_barrier()

    ebase = wid * EPW

    def issue_idx(p, g):
        base = ebase + g * B
        pltpu.async_copy(src_hbm.at[pl.ds(base, B)], srcs[p], isems[p])
        pltpu.async_copy(dst_hbm.at[pl.ds(base, B)], dsts[p], isems[p])
        pltpu.async_copy(ty_hbm.at[pl.ds(base, B)], tys[p], isems[p])

    def wait_idx(p):
        pltpu.make_async_copy(src_hbm.at[pl.ds(0, B)], srcs[p], isems[p]).wait()
        pltpu.make_async_copy(dst_hbm.at[pl.ds(0, B)], dsts[p], isems[p]).wait()
        pltpu.make_async_copy(ty_hbm.at[pl.ds(0, B)], tys[p], isems[p]).wait()

    def issue_gather(p):
        pltpu.async_copy(qq_hbm.at[dsts[p]], qqs[p], gsems[p])
        pltpu.async_copy(kv_hbm.at[srcs[p]], kvs[p], gsems[p])

    def wait_gather(p):
        pltpu.make_async_copy(qq_hbm.at[dsts[p]], qqs[p], gsems[p]).wait()
        pltpu.make_async_copy(kv_hbm.at[srcs[p]], kvs[p], gsems[p]).wait()

    def issue_scatter(p):
        pltpu.async_copy(srows[p], agg_s.at[sdsts[p]], ssems[p], add=True)

    def wait_scatter(p):
        pltpu.make_async_copy(srows[p], agg_s.at[sdsts[p]], ssems[p]).wait()

    def compute(p, nege):
        qq_v, kv_v, srow_v = qqs[p], kvs[p], srows[p]

        def group(eg, carry2):
            evec = eg * 16 + lane
            tvec = tys[p][pl.ds(eg * 16, 16)]
            accs = [plsc.load_gather(qq_v, [evec, D + tvec * H + hh])
                    for hh in range(H)]
            dv = jnp.zeros((16,), jnp.int32)
            for dcol in range(D):
                qv = plsc.load_gather(qq_v, [evec, dv])
                kvv = plsc.load_gather(kv_v, [evec, dv])
                accs[dcol // DH] = accs[dcol // DH] + qv * kvv
                dv = dv + one
            ws = [jnp.exp(a) for a in accs]
            for hh in range(H):
                plsc.store_scatter(srow_v, [evec, jnp.full((16,), D + hh, jnp.int32)],
                                   ws[hh])
            sv = jnp.zeros((16,), jnp.int32)
            for dcol in range(D):
                vv = (plsc.load_gather(kv_v, [evec, sv + D])
                      + plsc.load_gather(ev_v, [tvec, sv]))
                plsc.store_scatter(srow_v, [evec, sv], ws[dcol // DH] * vv)
                sv = sv + one
            sdsts[p][pl.ds(eg * 16, 16)] = dsts[p][pl.ds(eg * 16, 16)]
            return carry2

        lax.fori_loop(0, nege, group, 0)

    issue_idx(0, 0)
    wait_idx(0)
    issue_gather(0)
    issue_idx(1, 1)

    def step(g, p):
        q = 1 - p
        wait_gather(p)
        compute(p, G)
        @pl.when(g + 1 < NBLK)
        def _():
            wait_idx(q)
            issue_gather(q)
        @pl.when(g > 0)
        def _():
            wait_scatter(q)
        issue_scatter(p)
        @pl.when(g + 2 < NBLK)
        def _():
            issue_idx(p, g + 2)

    def dstep(t, carry):
        step(2 * t, 0)
        step(2 * t + 1, 1)
        return carry

    lax.fori_loop(0, NBLK // 2, dstep, 0)
    step(NBLK - 1, 0)
    wait_scatter(0)

    plsc.subcore_barrier()

    @pl.when(s < CPT)
    def _():
        pltpu.sync_copy(agg_s.at[pl.ds(s * ROWS_T, ROWS_T), :],
                        agg_out.at[c, pl.ds(s * ROWS_T, ROWS_T), :])


_sc_edge = functools.partial(
    pl.kernel,
    out_type=jax.ShapeDtypeStruct((NC, N, SW), jnp.float32),
    mesh=plsc.VectorSubcoreMesh(core_axis_name="c", subcore_axis_name="s",
                                num_cores=NC),
    compiler_params=pltpu.CompilerParams(use_tc_tiling_on_sc=False,
                                         needs_layout_passes=False),
    scratch_types=[
        pltpu.VMEM((B,), jnp.int32),
        pltpu.VMEM((B,), jnp.int32),
        pltpu.VMEM((B,), jnp.int32),
        pltpu.VMEM((B,), jnp.int32),
        pltpu.VMEM((B,), jnp.int32),
        pltpu.VMEM((B,), jnp.int32),
        pltpu.VMEM((B,), jnp.int32),
        pltpu.VMEM((B,), jnp.int32),
        pltpu.VMEM((B, D + NT * H), jnp.float32),
        pltpu.VMEM((B, D + NT * H), jnp.float32),
        pltpu.VMEM((B, 2 * D), jnp.float32),
        pltpu.VMEM((B, 2 * D), jnp.float32),
        pltpu.VMEM((B, SW), jnp.float32),
        pltpu.VMEM((B, SW), jnp.float32),
        pltpu.VMEM((NT, D), jnp.float32),
        pltpu.VMEM_SHARED((N, SW), jnp.float32),
        pltpu.SemaphoreType.DMA,
        pltpu.SemaphoreType.DMA,
        pltpu.SemaphoreType.DMA,
        pltpu.SemaphoreType.DMA,
        pltpu.SemaphoreType.DMA,
        pltpu.SemaphoreType.DMA,
    ],
)(_sc_body)


def kernel(x, edge_index, edge_type, params):
    src = edge_index[0]
    dst = edge_index[1]
    ty = edge_type
    heads = jnp.arange(D) // DH
    hid = jnp.arange(H)
    E4 = (jnp.arange(16)[:, None] == heads[None, :]).astype(jnp.float32)

    h = x
    for p in params['layers']:
        scale = 1.0 / np.sqrt(DH)
        EkT = params['edge_table'] @ p['Wek']
        EvT = params['edge_table'] @ p['Wev']
        M = (EkT.T[:, :, None]
             * (heads[:, None, None] == hid[None, None, :])).reshape(D, NT * H)
        Wq_s = p['Wq'] * scale
        Wa = jnp.concatenate([Wq_s, Wq_s @ M], axis=1)
        Wb = jnp.concatenate([p['Wk'], p['Wv']], axis=1)

        qq, kv = _tc_proj(h, Wa, Wb)
        agg2 = _sc_edge(qq, kv, EvT, src, dst, ty)
        h = _tc_merge(h, agg2, E4,
                      p['Wn'][:D], p['Wn'][D:],
                      p['bn'].reshape(1, D),
                      p['ln_s'].reshape(1, D),
                      p['ln_b'].reshape(1, D))
    return h.reshape(N, 1, D)


# speedup vs baseline: 10.2989x; 1.1943x over previous
import functools

import jax
import jax.numpy as jnp
import numpy as np
from jax import lax
from jax.experimental import pallas as pl
from jax.experimental.pallas import tpu as pltpu
from jax.experimental.pallas import tpu_sc as plsc

N = 10000
E = 320000
D = 128
H = 4
DH = D // H
NT = 16
NC = 2
NS = 16
NW = NC * NS
EPW = E // NW
B = 16
G = B // 16
NBLK = EPW // B
SB = 32
SBE = SB * B
EPAD = E + 2 * SBE
SW = D + 16
BN = 1000
CPT = 10
ROWS_T = N // CPT


def _tc_proj(h, Wa, Wb):
    def body(x_ref, wa_ref, wb_ref, qq_ref, kv_ref):
        x = x_ref[...]
        qq_ref[...] = jnp.dot(x, wa_ref[...], preferred_element_type=jnp.float32)
        kv_ref[...] = jnp.dot(x, wb_ref[...], preferred_element_type=jnp.float32)

    return pl.pallas_call(
        body,
        grid=(N // BN,),
        in_specs=[
            pl.BlockSpec((BN, D), lambda i: (i, 0)),
            pl.BlockSpec((D, D + NT * H), lambda i: (0, 0)),
            pl.BlockSpec((D, 2 * D), lambda i: (0, 0)),
        ],
        out_specs=[
            pl.BlockSpec((BN, D + NT * H), lambda i: (i, 0)),
            pl.BlockSpec((BN, 2 * D), lambda i: (i, 0)),
        ],
        out_shape=[
            jax.ShapeDtypeStruct((N, D + NT * H), jnp.float32),
            jax.ShapeDtypeStruct((N, 2 * D), jnp.float32),
        ],
    )(h, Wa, Wb)


def _tc_merge(h, agg2, E4, WnT, WnB, bn, ln_s, ln_b):
    def body(x_ref, a_ref, e4_ref, wnt_ref, wnb_ref,
             bn_ref, lns_ref, lnb_ref, o_ref):
        ab = a_ref[0] + a_ref[1]
        A = ab[:, :D]
        denx = jnp.dot(ab[:, D:], e4_ref[...],
                       preferred_element_type=jnp.float32)
        aggn = A / (denx + 1e-9)
        z = (jnp.dot(x_ref[...], wnt_ref[...], preferred_element_type=jnp.float32)
             + jnp.dot(aggn, wnb_ref[...], preferred_element_type=jnp.float32)
             + bn_ref[...])
        z = jnp.maximum(z, 0.0)
        mu = jnp.mean(z, axis=1, keepdims=True)
        var = jnp.mean((z - mu) ** 2, axis=1, keepdims=True)
        o_ref[...] = (z - mu) / jnp.sqrt(var + 1e-5) * lns_ref[...] + lnb_ref[...]

    return pl.pallas_call(
        body,
        grid=(N // BN,),
        in_specs=[
            pl.BlockSpec((BN, D), lambda i: (i, 0)),
            pl.BlockSpec((NC, BN, SW), lambda i: (0, i, 0)),
            pl.BlockSpec((16, D), lambda i: (0, 0)),
            pl.BlockSpec((D, D), lambda i: (0, 0)),
            pl.BlockSpec((D, D), lambda i: (0, 0)),
            pl.BlockSpec((1, D), lambda i: (0, 0)),
            pl.BlockSpec((1, D), lambda i: (0, 0)),
            pl.BlockSpec((1, D), lambda i: (0, 0)),
        ],
        out_specs=pl.BlockSpec((BN, D), lambda i: (i, 0)),
        out_shape=jax.ShapeDtypeStruct((N, D), jnp.float32),
    )(h, agg2, E4, WnT, WnB, bn, ln_s, ln_b)


def _sc_body(qq_hbm, kv_hbm, ev_hbm, src_hbm, dst_hbm, ty_hbm,
             agg_out,
             sbs_v0, sbs_v1, sbd_v0, sbd_v1, sbt_v0, sbt_v1,
             sdst_v0, sdst_v1,
             qq_v0, qq_v1, kv_v0, kv_v1,
             srow_v0, srow_v1, ev_v,
             agg_s,
             rsem0, rsem1, gsem0, gsem1, ssem0, ssem1):
    c = lax.axis_index("c")
    s = lax.axis_index("s")
    wid = c * NS + s
    zvec = jnp.zeros((16,), jnp.float32)
    lane = lax.iota(jnp.int32, 16)
    one = jnp.ones((16,), jnp.int32)

    sbss = (sbs_v0, sbs_v1)
    sbds = (sbd_v0, sbd_v1)
    sbts = (sbt_v0, sbt_v1)
    sdsts = (sdst_v0, sdst_v1)
    qqs = (qq_v0, qq_v1)
    kvs = (kv_v0, kv_v1)
    srows = (srow_v0, srow_v1)
    rsems = (rsem0, rsem1)
    gsems = (gsem0, gsem1)
    ssems = (ssem0, ssem1)

    pltpu.sync_copy(ev_hbm, ev_v)

    @pl.when(s < CPT)
    def _():
        def zrow(i, cy):
            for j in range(SW // 16):
                srow_v0[i, pl.ds(j * 16, 16)] = zvec
            return cy
        lax.fori_loop(0, B, zrow, 0)
        base = s * ROWS_T
        def zcpy(i, cy):
            pltpu.sync_copy(srow_v0, agg_s.at[pl.ds(base + i * B, B), :])
            return cy
        lax.fori_loop(0, ROWS_T // B, zcpy, 0)
        rem = ROWS_T % B
        pltpu.sync_copy(srow_v0.at[pl.ds(0, rem), :],
                        agg_s.at[pl.ds(base + (ROWS_T // B) * B, rem), :])

    plsc.subcore_barrier()

    ebase = wid * EPW

    def issue_refill(sb, rp):
        base = ebase + sb * SBE
        pltpu.async_copy(src_hbm.at[pl.ds(base, SBE)], sbss[rp], rsems[rp])
        pltpu.async_copy(dst_hbm.at[pl.ds(base, SBE)], sbds[rp], rsems[rp])
        pltpu.async_copy(ty_hbm.at[pl.ds(base, SBE)], sbts[rp], rsems[rp])

    def wait_refill(rp):
        pltpu.make_async_copy(src_hbm.at[pl.ds(0, SBE)], sbss[rp], rsems[rp]).wait()
        pltpu.make_async_copy(dst_hbm.at[pl.ds(0, SBE)], sbds[rp], rsems[rp]).wait()
        pltpu.make_async_copy(ty_hbm.at[pl.ds(0, SBE)], sbts[rp], rsems[rp]).wait()

    def issue_gather(g, p):
        rp = lax.rem(g // SB, 2)
        o = lax.rem(g, SB) * B
        @pl.when(rp == 0)
        def _():
            pltpu.async_copy(qq_hbm.at[sbds[0].at[pl.ds(o, B)]], qqs[p], gsems[p])
            pltpu.async_copy(kv_hbm.at[sbss[0].at[pl.ds(o, B)]], kvs[p], gsems[p])
        @pl.when(rp == 1)
        def _():
            pltpu.async_copy(qq_hbm.at[sbds[1].at[pl.ds(o, B)]], qqs[p], gsems[p])
            pltpu.async_copy(kv_hbm.at[sbss[1].at[pl.ds(o, B)]], kvs[p], gsems[p])

    def wait_gather(p):
        pltpu.make_async_copy(qq_hbm.at[sbds[0].at[pl.ds(0, B)]], qqs[p], gsems[p]).wait()
        pltpu.make_async_copy(kv_hbm.at[sbss[0].at[pl.ds(0, B)]], kvs[p], gsems[p]).wait()

    def issue_scatter(p):
        pltpu.async_copy(srows[p], agg_s.at[sdsts[p]], ssems[p], add=True)

    def wait_scatter(p):
        pltpu.make_async_copy(srows[p], agg_s.at[sdsts[p]], ssems[p]).wait()

    def compute(g, p):
        qq_v, kv_v, srow_v = qqs[p], kvs[p], srows[p]
        rp = lax.rem(g // SB, 2)
        o = lax.rem(g, SB) * B
        dvec0 = lax.select(rp == 0, sbds[0][pl.ds(o, 16)], sbds[1][pl.ds(o, 16)])
        tvec = lax.select(rp == 0, sbts[0][pl.ds(o, 16)], sbts[1][pl.ds(o, 16)])
        evec = lane
        accs = [plsc.load_gather(qq_v, [evec, D + tvec * H + hh])
                for hh in range(H)]
        dv = jnp.zeros((16,), jnp.int32)
        for dcol in range(D):
            qv = plsc.load_gather(qq_v, [evec, dv])
            kvv = plsc.load_gather(kv_v, [evec, dv])
            accs[dcol // DH] = accs[dcol // DH] + qv * kvv
            dv = dv + one
        ws = [jnp.exp(a) for a in accs]
        for hh in range(H):
            plsc.store_scatter(srow_v, [evec, jnp.full((16,), D + hh, jnp.int32)],
                               ws[hh])
        sv = jnp.zeros((16,), jnp.int32)
        for dcol in range(D):
            vv = (plsc.load_gather(kv_v, [evec, sv + D])
                  + plsc.load_gather(ev_v, [tvec, sv]))
            plsc.store_scatter(srow_v, [evec, sv], ws[dcol // DH] * vv)
            sv = sv + one
        sdsts[p][pl.ds(0, 16)] = dvec0

    issue_refill(0, 0)
    wait_refill(0)
    issue_refill(1, 1)
    issue_gather(0, 0)
    issue_gather(1, 1)

    def step(g, p):
        q = 1 - p
        @pl.when(jnp.logical_and(
            jnp.logical_and(lax.rem(g, SB) == 0, g > 0),
            (g // SB + 1) * SB < NBLK))
        def _():
            sbi = g // SB + 1
            @pl.when(lax.rem(sbi, 2) == 0)
            def _():
                issue_refill(sbi, 0)
            @pl.when(lax.rem(sbi, 2) == 1)
            def _():
                issue_refill(sbi, 1)
        @pl.when(jnp.logical_and(lax.rem(g + 1, SB) == 0, g + 1 < NBLK))
        def _():
            sbn = (g + 1) // SB
            @pl.when(lax.rem(sbn, 2) == 0)
            def _():
                wait_refill(0)
            @pl.when(lax.rem(sbn, 2) == 1)
            def _():
                wait_refill(1)
        @pl.when(g + 1 < NBLK)
        def _():
            issue_gather(g + 1, q)
        wait_gather(p)
        @pl.when(g >= 2)
        def _():
            wait_scatter(p)
        compute(g, p)
        issue_scatter(p)

    def dstep(t, carry):
        step(2 * t, 0)
        step(2 * t + 1, 1)
        return carry

    lax.fori_loop(0, NBLK // 2, dstep, 0)
    step(NBLK - 1, 0)
    wait_scatter(1)
    wait_scatter(0)

    plsc.subcore_barrier()

    @pl.when(s < CPT)
    def _():
        pltpu.sync_copy(agg_s.at[pl.ds(s * ROWS_T, ROWS_T), :],
                        agg_out.at[c, pl.ds(s * ROWS_T, ROWS_T), :])


_sc_edge = functools.partial(
    pl.kernel,
    out_type=jax.ShapeDtypeStruct((NC, N, SW), jnp.float32),
    mesh=plsc.VectorSubcoreMesh(core_axis_name="c", subcore_axis_name="s",
                                num_cores=NC),
    compiler_params=pltpu.CompilerParams(use_tc_tiling_on_sc=False,
                                         needs_layout_passes=False),
    scratch_types=[
        pltpu.VMEM((SBE,), jnp.int32),
        pltpu.VMEM((SBE,), jnp.int32),
        pltpu.VMEM((SBE,), jnp.int32),
        pltpu.VMEM((SBE,), jnp.int32),
        pltpu.VMEM((SBE,), jnp.int32),
        pltpu.VMEM((SBE,), jnp.int32),
        pltpu.VMEM((B,), jnp.int32),
        pltpu.VMEM((B,), jnp.int32),
        pltpu.VMEM((B, D + NT * H), jnp.float32),
        pltpu.VMEM((B, D + NT * H), jnp.float32),
        pltpu.VMEM((B, 2 * D), jnp.float32),
        pltpu.VMEM((B, 2 * D), jnp.float32),
        pltpu.VMEM((B, SW), jnp.float32),
        pltpu.VMEM((B, SW), jnp.float32),
        pltpu.VMEM((NT, D), jnp.float32),
        pltpu.VMEM_SHARED((N, SW), jnp.float32),
        pltpu.SemaphoreType.DMA,
        pltpu.SemaphoreType.DMA,
        pltpu.SemaphoreType.DMA,
        pltpu.SemaphoreType.DMA,
        pltpu.SemaphoreType.DMA,
        pltpu.SemaphoreType.DMA,
    ],
)(_sc_body)


def kernel(x, edge_index, edge_type, params):
    pad = jnp.zeros((EPAD - E,), edge_index.dtype)
    src = jnp.concatenate([edge_index[0], pad])
    dst = jnp.concatenate([edge_index[1], pad])
    ty = jnp.concatenate([edge_type, pad])
    heads = jnp.arange(D) // DH
    hid = jnp.arange(H)
    E4 = (jnp.arange(16)[:, None] == heads[None, :]).astype(jnp.float32)

    h = x
    for p in params['layers']:
        scale = 1.0 / np.sqrt(DH)
        EkT = params['edge_table'] @ p['Wek']
        EvT = params['edge_table'] @ p['Wev']
        M = (EkT.T[:, :, None]
             * (heads[:, None, None] == hid[None, None, :])).reshape(D, NT * H)
        Wq_s = p['Wq'] * scale
        Wa = jnp.concatenate([Wq_s, Wq_s @ M], axis=1)
        Wb = jnp.concatenate([p['Wk'], p['Wv']], axis=1)

        qq, kv = _tc_proj(h, Wa, Wb)
        agg2 = _sc_edge(qq, kv, EvT, src, dst, ty)
        h = _tc_merge(h, agg2, E4,
                      p['Wn'][:D], p['Wn'][D:],
                      p['bn'].reshape(1, D),
                      p['ln_s'].reshape(1, D),
                      p['ln_b'].reshape(1, D))
    return h.reshape(N, 1, D)
